# Initial kernel scaffold; baseline (speedup 1.0000x reference)
import functools

import jax
import jax.numpy as jnp
from jax import lax
from jax.experimental import pallas as pl
from jax.experimental.pallas import tpu as pltpu
from jax.experimental.pallas import tpu_sc as plsc

V = 100002
D = 128
B = 4096
L = 50
LP = 64
NC = 2
NS = 16
NW = NC * NS
BPW = B // NW
CH = 8
NCHUNK = BPW // CH
NK = D // 16


def _sc_kernel_body(idx_hbm, w_hbm, table_hbm, out_hbm,
                    idx_v, w_v, rows_v, out_v, sem):
    cid = lax.axis_index("c")
    sid = lax.axis_index("s")
    wid = sid * NC + cid
    base = wid * BPW

    pltpu.sync_copy(idx_hbm.at[pl.ds(base, BPW)], idx_v)
    pltpu.sync_copy(w_hbm.at[pl.ds(base, BPW)], w_v)

    def chunk_body(c, _):
        for b in range(CH):
            pltpu.async_copy(table_hbm.at[idx_v.at[c * CH + b]],
                             rows_v.at[b], sem)
        for b in range(CH):
            pltpu.make_async_copy(table_hbm.at[idx_v.at[c * CH + b]],
                                  rows_v.at[b], sem).wait()

        for b in range(CH):
            row = c * CH + b
            wtot = (w_v[row, pl.ds(0, 16)] + w_v[row, pl.ds(16, 16)]
                    + w_v[row, pl.ds(32, 16)] + w_v[row, pl.ds(48, 16)])
            ws = jnp.maximum(jnp.sum(wtot), jnp.float32(1e-9))
            inv = jnp.float32(1.0) / jnp.full((16,), ws, jnp.float32)

            row_splat = jnp.full((16,), row, jnp.int32)

            def tok_body(l, accs):
                wbc = plsc.load_gather(
                    w_v, [row_splat, jnp.full((16,), l, jnp.int32)])
                return tuple(
                    accs[k] + wbc * rows_v[b, l, pl.ds(k * 16, 16)]
                    for k in range(NK))

            accs = lax.fori_loop(
                0, L, tok_body,
                tuple(jnp.zeros((16,), jnp.float32) for _ in range(NK)))

            for k in range(NK):
                out_v[b, pl.ds(k * 16, 16)] = accs[k] * inv

        pltpu.sync_copy(out_v, out_hbm.at[pl.ds(base + c * CH, CH)])
        return 0

    lax.fori_loop(0, NCHUNK, chunk_body, 0)


def kernel(token_indices, tfidf_weights, embedding_weight):
    mesh = plsc.VectorSubcoreMesh(core_axis_name="c", subcore_axis_name="s")
    run = functools.partial(
        pl.kernel,
        mesh=mesh,
        out_type=jax.ShapeDtypeStruct((B, D), jnp.float32),
        scratch_types=[
            pltpu.VMEM((BPW, L), jnp.int32),
            pltpu.VMEM((BPW, LP), jnp.float32),
            pltpu.VMEM((CH, L, D), jnp.float32),
            pltpu.VMEM((CH, D), jnp.float32),
            pltpu.SemaphoreType.DMA,
        ],
    )(_sc_kernel_body)
    idx = token_indices.astype(jnp.int32)
    w_pad = jnp.pad(tfidf_weights.astype(jnp.float32), ((0, 0), (0, LP - L)))
    return run(idx, w_pad, embedding_weight.astype(jnp.float32))


# speedup vs baseline: 9.2079x; 9.2079x over previous
import functools

import jax
import jax.numpy as jnp
from jax import lax
from jax.experimental import pallas as pl
from jax.experimental.pallas import tpu as pltpu
from jax.experimental.pallas import tpu_sc as plsc

V = 100002
D = 128
B = 4096
L = 50
LP = 64
NC = 2
NS = 16
NW = NC * NS
BPW = B // NW
CH = 8
NCHUNK = BPW // CH
NK = D // 16


_SPLAT_DNUMS = lax.GatherDimensionNumbers(
    offset_dims=(), collapsed_slice_dims=(0,), start_index_map=(0,))


def _splat_lane(vec, j):
    idx = jnp.full((16, 1), j, jnp.int32)
    return lax.gather(vec, idx, _SPLAT_DNUMS, (1,),
                      mode=lax.GatherScatterMode.PROMISE_IN_BOUNDS)


def _sc_kernel_body(idx_hbm, w_hbm, table_hbm, out_hbm,
                    idx_v, w_v, rows_v, out_v, sem):
    cid = lax.axis_index("c")
    sid = lax.axis_index("s")
    wid = sid * NC + cid
    base = wid * BPW

    pltpu.sync_copy(idx_hbm.at[pl.ds(base, BPW)], idx_v)
    pltpu.sync_copy(w_hbm.at[pl.ds(base, BPW)], w_v)

    def chunk_body(c, _):
        for b in range(CH):
            pltpu.async_copy(table_hbm.at[idx_v.at[c * CH + b]],
                             rows_v.at[b], sem)
        for b in range(CH):
            pltpu.make_async_copy(table_hbm.at[idx_v.at[c * CH + b]],
                                  rows_v.at[b], sem).wait()

        for b in range(CH):
            row = c * CH + b
            accs = tuple(jnp.zeros((16,), jnp.float32) for _ in range(NK))
            wacc = jnp.zeros((16,), jnp.float32)

            for g in range(LP // 16):
                wvec = w_v[row, pl.ds(g * 16, 16)]
                nt = min(16, L - g * 16)

                def tok_body(j, carry, wvec=wvec, g=g):
                    accs, wacc = carry
                    wbc = _splat_lane(wvec, j)
                    t = g * 16 + j
                    new = tuple(
                        accs[k] + wbc * rows_v[b, t, pl.ds(k * 16, 16)]
                        for k in range(NK))
                    return new, wacc + wbc

                accs, wacc = lax.fori_loop(0, nt, tok_body, (accs, wacc))

            inv = jnp.float32(1.0) / jnp.maximum(wacc, jnp.float32(1e-9))
            for k in range(NK):
                out_v[b, pl.ds(k * 16, 16)] = accs[k] * inv

        pltpu.sync_copy(out_v, out_hbm.at[pl.ds(base + c * CH, CH)])
        return 0

    lax.fori_loop(0, NCHUNK, chunk_body, 0)


def kernel(token_indices, tfidf_weights, embedding_weight):
    mesh = plsc.VectorSubcoreMesh(core_axis_name="c", subcore_axis_name="s")
    run = functools.partial(
        pl.kernel,
        mesh=mesh,
        out_type=jax.ShapeDtypeStruct((B, D), jnp.float32),
        scratch_types=[
            pltpu.VMEM((BPW, L), jnp.int32),
            pltpu.VMEM((BPW, LP), jnp.float32),
            pltpu.VMEM((CH, L, D), jnp.float32),
            pltpu.VMEM((CH, D), jnp.float32),
            pltpu.SemaphoreType.DMA,
        ],
    )(_sc_kernel_body)
    idx = token_indices.astype(jnp.int32)
    w_pad = jnp.pad(tfidf_weights.astype(jnp.float32), ((0, 0), (0, LP - L)))
    return run(idx, w_pad, embedding_weight.astype(jnp.float32))
